# baseline (speedup 1.0000x reference)
import jax
import jax.numpy as jnp
from jax import lax
from jax.experimental import pallas as pl
from jax.experimental.pallas import tpu as pltpu

B = 128
T = 4096
NCH = 8
CR = B // NCH


def _col(sm, base, n):
    return jnp.concatenate(
        [jnp.full((1, 1), sm[base + r], jnp.int32) for r in range(n)], axis=0)


def _body(starts_sm, ends_sm, nf_sm, oidx_sm, e_any, out_any, *scratch):
    bufs = scratch[:NCH]
    fsem = scratch[NCH]
    rsem = scratch[NCH + 1]
    oidx = oidx_sm[0]
    pos = lax.broadcasted_iota(jnp.int32, (CR, T), 1)

    for c in range(NCH):
        base = c * CR
        scol = _col(starts_sm, base, CR)
        ecol = _col(ends_sm, base, CR)
        in_win = (pos >= scol) & (pos < ecol)
        bufs[c][...] = jnp.where(in_win, jnp.float32(1.0), jnp.float32(-99999.0))
        pltpu.make_async_copy(
            bufs[c], out_any.at[pl.ds(base, CR)], fsem.at[c]).start()

    for c in range(NCH):
        base = c * CR
        pltpu.make_async_copy(
            bufs[c], out_any.at[pl.ds(base, CR)], fsem.at[c]).wait()
        for r in range(CR):
            @pl.when(oidx >= nf_sm[base + r])
            def _():
                pltpu.make_async_copy(
                    e_any.at[base + r], out_any.at[base + r], rsem).start()

    k = lax.fori_loop(
        0, B,
        lambda r, acc: acc + (oidx >= nf_sm[r]).astype(jnp.int32),
        jnp.int32(0))

    def drain(_, carry):
        pltpu.make_async_copy(e_any.at[0], out_any.at[0], rsem).wait()
        return carry

    lax.fori_loop(0, k, drain, 0)


@jax.jit
def _tc_kernel(starts, ends, nf, oidx, e):
    grid_spec = pltpu.PrefetchScalarGridSpec(
        num_scalar_prefetch=4,
        grid=(1,),
        in_specs=[pl.BlockSpec(memory_space=pl.ANY)],
        out_specs=pl.BlockSpec(memory_space=pl.ANY),
        scratch_shapes=([pltpu.VMEM((CR, T), jnp.float32)] * NCH
                        + [pltpu.SemaphoreType.DMA((NCH,)),
                           pltpu.SemaphoreType.DMA]),
    )
    return pl.pallas_call(
        _body,
        grid_spec=grid_spec,
        out_shape=jax.ShapeDtypeStruct((B, T), jnp.float32),
    )(starts, ends, nf, oidx, e)


def kernel(e, att_starts, att_ends, n_att_frames, output_index):
    oidx = jnp.asarray(output_index, jnp.int32).reshape(1)
    return _tc_kernel(att_starts.astype(jnp.int32), att_ends.astype(jnp.int32),
                      n_att_frames.astype(jnp.int32), oidx, e)


# speedup vs baseline: 3.6149x; 3.6149x over previous
import jax
import jax.numpy as jnp
from jax import lax
from jax.experimental import pallas as pl
from jax.experimental.pallas import tpu as pltpu

B = 128
T = 4096
BR = 64
NBLK = B // BR


def _body(starts_sm, ends_sm, nf_sm, oidx_sm, e_ref, out_ref, cols):
    g = pl.program_id(0)

    @pl.when(g == 0)
    def _():
        svals = [jnp.full((1, 1), starts_sm[r], jnp.int32) for r in range(B)]
        evals = [jnp.full((1, 1), ends_sm[r], jnp.int32) for r in range(B)]
        fvals = [jnp.full((1, 1), nf_sm[r], jnp.int32) for r in range(B)]
        scol = jnp.concatenate(svals, axis=0)
        ecol = jnp.concatenate(evals, axis=0)
        fcol = jnp.concatenate(fvals, axis=0)
        cols[:, 0:1] = scol
        cols[:, 1:2] = ecol - scol
        cols[:, 2:3] = fcol

    base = g * BR
    scol = cols[pl.ds(base, BR), 0:1]
    lcol = cols[pl.ds(base, BR), 1:2]
    fcol = cols[pl.ds(base, BR), 2:3]
    pos = lax.broadcasted_iota(jnp.int32, (BR, T), 1)
    in_win = (pos - scol).astype(jnp.uint32) < lcol.astype(jnp.uint32)
    oracle = jnp.where(in_win, jnp.float32(1.0), jnp.float32(-99999.0))
    out_ref[...] = jnp.where(oidx_sm[0] < fcol, oracle, e_ref[...])


@jax.jit
def _tc_kernel(starts, ends, nf, oidx, e):
    grid_spec = pltpu.PrefetchScalarGridSpec(
        num_scalar_prefetch=4,
        grid=(NBLK,),
        in_specs=[pl.BlockSpec((BR, T), lambda i, *_: (i, 0))],
        out_specs=pl.BlockSpec((BR, T), lambda i, *_: (i, 0)),
        scratch_shapes=[pltpu.VMEM((B, 3), jnp.int32)],
    )
    return pl.pallas_call(
        _body,
        grid_spec=grid_spec,
        out_shape=jax.ShapeDtypeStruct((B, T), jnp.float32),
    )(starts, ends, nf, oidx, e)


def kernel(e, att_starts, att_ends, n_att_frames, output_index):
    oidx = jnp.asarray(output_index, jnp.int32).reshape(1)
    return _tc_kernel(att_starts.astype(jnp.int32), att_ends.astype(jnp.int32),
                      n_att_frames.astype(jnp.int32), oidx, e)


# speedup vs baseline: 3.6482x; 1.0092x over previous
import jax
import jax.numpy as jnp
from jax import lax
from jax.experimental import pallas as pl
from jax.experimental.pallas import tpu as pltpu

B = 128
T = 4096
BR = 64
NBLK = B // BR


def _body(starts_sm, ends_sm, nf_sm, oidx_sm, e_ref, out_ref, cols):
    g = pl.program_id(0)

    @pl.when(g == 0)
    def _():
        svals = [jnp.full((1, 1), starts_sm[r], jnp.int32) for r in range(B)]
        evals = [jnp.full((1, 1), ends_sm[r], jnp.int32) for r in range(B)]
        fvals = [jnp.full((1, 1), nf_sm[r], jnp.int32) for r in range(B)]
        scol = jnp.concatenate(svals, axis=0)
        ecol = jnp.concatenate(evals, axis=0)
        fcol = jnp.concatenate(fvals, axis=0)
        cols[:, 0:1] = scol
        cols[:, 1:2] = jnp.maximum(ecol - scol, 0)
        cols[:, 2:3] = fcol

    base = g * BR
    scol = cols[pl.ds(base, BR), 0:1]
    lcol = cols[pl.ds(base, BR), 1:2]
    fcol = cols[pl.ds(base, BR), 2:3]
    pos = lax.broadcasted_iota(jnp.int32, (BR, T), 1)
    in_win = (pos - scol).astype(jnp.uint32) < lcol.astype(jnp.uint32)
    oracle = jnp.where(in_win, jnp.float32(1.0), jnp.float32(-99999.0))
    out_ref[...] = jnp.where(oidx_sm[0] < fcol, oracle, e_ref[...])


@jax.jit
def _tc_kernel(starts, ends, nf, oidx, e):
    grid_spec = pltpu.PrefetchScalarGridSpec(
        num_scalar_prefetch=4,
        grid=(NBLK,),
        in_specs=[pl.BlockSpec((BR, T), lambda i, *_: (i, 0))],
        out_specs=pl.BlockSpec((BR, T), lambda i, *_: (i, 0)),
        scratch_shapes=[pltpu.VMEM((B, 3), jnp.int32)],
    )
    return pl.pallas_call(
        _body,
        grid_spec=grid_spec,
        out_shape=jax.ShapeDtypeStruct((B, T), jnp.float32),
    )(starts, ends, nf, oidx, e)


def kernel(e, att_starts, att_ends, n_att_frames, output_index):
    oidx = jnp.asarray(output_index, jnp.int32).reshape(1)
    return _tc_kernel(att_starts.astype(jnp.int32), att_ends.astype(jnp.int32),
                      n_att_frames.astype(jnp.int32), oidx, e)
